# Initial kernel scaffold; baseline (speedup 1.0000x reference)
import jax
import jax.numpy as jnp
from jax.experimental import pallas as pl


def kernel(x, edge_index, edge_weight, batch, W1, b1, W2, b2, W3, b3, Wl, bl):
    raise NotImplementedError("write your pallas kernel here")


# speedup vs baseline: 8.0444x; 8.0444x over previous
import functools

import jax
import jax.numpy as jnp
from jax import lax
from jax.experimental import pallas as pl
from jax.experimental.pallas import tpu as pltpu
from jax.experimental.pallas import tpu_sc as plsc

N = 10000
E = 160000
G = 64
D_IN = 1056
H1, H2, H3, D_OUT = 256, 128, 64, 3

NC, NS = 2, 16
CH = 125
EC_AGG = E // NS
NCH_AGG = EC_AGG // CH
EC_DEG = E // (NS * NC)
NCH_DEG = EC_DEG // CH
ROWS_FULL = 640
ROWS_LAST = N - (NS - 1) * ROWS_FULL
ZROWS = 80

MB = 512
GRID = (N + MB - 1) // MB

_mesh = plsc.VectorSubcoreMesh(
    core_axis_name="c", subcore_axis_name="s", num_cores=NC, num_subcores=NS)
_sc_params = pltpu.CompilerParams(
    needs_layout_passes=False, use_tc_tiling_on_sc=False)


def _deg_body(c_hbm, w_hbm, out_hbm, c_v, w_v, zb, deg_sh):
    cid = lax.axis_index("c")
    sid = lax.axis_index("s")
    wid = cid * NS + sid
    pltpu.sync_copy(c_hbm.at[wid], c_v)
    pltpu.sync_copy(w_hbm.at[wid], w_v)
    for i in range(ROWS_FULL // 16):
        zb[pl.ds(i * 16, 16)] = jnp.zeros((16,), jnp.float32)
    row0 = sid * ROWS_FULL

    @pl.when(sid < NS - 1)
    def _():
        pltpu.sync_copy(zb, deg_sh.at[pl.ds(row0, ROWS_FULL)])

    @pl.when(sid == NS - 1)
    def _():
        pltpu.sync_copy(zb.at[pl.ds(0, ROWS_LAST)], deg_sh.at[pl.ds(row0, ROWS_LAST)])

    plsc.subcore_barrier()

    @pl.loop(0, NCH_DEG)
    def _scatter(j):
        pltpu.sync_copy(w_v.at[j], deg_sh.at[c_v.at[j]], add=True)

    plsc.subcore_barrier()
    base = cid * N + row0

    @pl.when(sid < NS - 1)
    def _():
        pltpu.sync_copy(deg_sh.at[pl.ds(row0, ROWS_FULL)], zb)
        pltpu.sync_copy(zb, out_hbm.at[pl.ds(base, ROWS_FULL)])

    @pl.when(sid == NS - 1)
    def _():
        pltpu.sync_copy(deg_sh.at[pl.ds(row0, ROWS_LAST)], zb.at[pl.ds(0, ROWS_LAST)])
        pltpu.sync_copy(zb.at[pl.ds(0, ROWS_LAST)], out_hbm.at[pl.ds(base, ROWS_LAST)])


_deg_call = pl.kernel(
    _deg_body,
    out_type=jax.ShapeDtypeStruct((NC * N,), jnp.float32),
    mesh=_mesh,
    compiler_params=_sc_params,
    scratch_types=[
        pltpu.VMEM((NCH_DEG, CH), jnp.int32),
        pltpu.VMEM((NCH_DEG, CH), jnp.float32),
        pltpu.VMEM((ROWS_FULL,), jnp.float32),
        pltpu.MemorySpace.VMEM_SHARED((N,), jnp.float32),
    ],
)


def _make_agg(dh):
    def body(xs_a, xs_b, r_hbm, c_hbm, w_hbm, out_a, out_b,
             r_v, c_v, w_v, buf, agg_sh):
        cid = lax.axis_index("c")
        sid = lax.axis_index("s")
        pltpu.sync_copy(r_hbm.at[sid], r_v)
        pltpu.sync_copy(c_hbm.at[sid], c_v)
        pltpu.sync_copy(w_hbm.at[sid], w_v)

        @pl.loop(0, CH)
        def _z(i):
            for k in range(dh // 16):
                buf[i, pl.ds(k * 16, 16)] = jnp.zeros((16,), jnp.float32)

        zblk = buf.at[pl.ds(0, ZROWS)]
        row0 = sid * ROWS_FULL

        @pl.when(sid < NS - 1)
        def _():
            @pl.loop(0, ROWS_FULL // ZROWS)
            def _c(i):
                pltpu.sync_copy(zblk, agg_sh.at[pl.ds(row0 + i * ZROWS, ZROWS)])

        @pl.when(sid == NS - 1)
        def _():
            @pl.loop(0, ROWS_LAST // ZROWS)
            def _c(i):
                pltpu.sync_copy(zblk, agg_sh.at[pl.ds(row0 + i * ZROWS, ZROWS)])

        plsc.subcore_barrier()

        def run(xs_hbm):
            @pl.loop(0, NCH_AGG)
            def _chunk(j):
                pltpu.sync_copy(xs_hbm.at[r_v.at[j]], buf)

                @pl.loop(0, CH)
                def _row(e):
                    fv = jnp.full((16,), j * CH + e, jnp.int32)
                    wv = plsc.load_gather(w_v, [fv])
                    for k in range(dh // 16):
                        sl = pl.ds(k * 16, 16)
                        buf[e, sl] = buf[e, sl] * wv

                pltpu.sync_copy(buf, agg_sh.at[c_v.at[j]], add=True)

        @pl.when(cid == 0)
        def _():
            run(xs_a)

        @pl.when(cid == 1)
        def _():
            run(xs_b)

        plsc.subcore_barrier()

        def drain(out_hbm):
            @pl.when(sid < NS - 1)
            def _():
                @pl.loop(0, ROWS_FULL // ZROWS)
                def _d(i):
                    pltpu.sync_copy(agg_sh.at[pl.ds(row0 + i * ZROWS, ZROWS)], zblk)
                    pltpu.sync_copy(zblk, out_hbm.at[pl.ds(row0 + i * ZROWS, ZROWS)])

            @pl.when(sid == NS - 1)
            def _():
                @pl.loop(0, ROWS_LAST // ZROWS)
                def _d(i):
                    pltpu.sync_copy(agg_sh.at[pl.ds(row0 + i * ZROWS, ZROWS)], zblk)
                    pltpu.sync_copy(zblk, out_hbm.at[pl.ds(row0 + i * ZROWS, ZROWS)])

        @pl.when(cid == 0)
        def _():
            drain(out_a)

        @pl.when(cid == 1)
        def _():
            drain(out_b)

    return pl.kernel(
        body,
        out_type=[jax.ShapeDtypeStruct((N, dh), jnp.float32)] * 2,
        mesh=_mesh,
        compiler_params=_sc_params,
        scratch_types=[
            pltpu.VMEM((NCH_AGG, CH), jnp.int32),
            pltpu.VMEM((NCH_AGG, CH), jnp.int32),
            pltpu.VMEM((EC_AGG,), jnp.float32),
            pltpu.VMEM((CH, dh), jnp.float32),
            pltpu.MemorySpace.VMEM_SHARED((N, dh), jnp.float32),
        ],
    )


_agg = {dh: _make_agg(dh) for dh in (H1 // 2, H2 // 2, H3 // 2)}


def _dis_of(deg_ref):
    d = deg_ref[...]
    s = d[:, 0:1] + d[:, 1:2] + 1.0
    return jnp.where(s > 0, lax.rsqrt(s), 0.0)


def _tc1(x, W1, degt):
    dh = H1 // 2

    def body(x_ref, w_ref, deg_ref, oa_ref, ob_ref):
        dis = _dis_of(deg_ref)
        xw = jnp.dot(x_ref[...], w_ref[...], preferred_element_type=jnp.float32)
        xs = xw * dis
        oa_ref[...] = xs[:, :dh]
        ob_ref[...] = xs[:, dh:]

    return pl.pallas_call(
        body,
        grid=(GRID,),
        in_specs=[
            pl.BlockSpec((MB, D_IN), lambda i: (i, 0)),
            pl.BlockSpec((D_IN, H1), lambda i: (0, 0)),
            pl.BlockSpec((MB, 2), lambda i: (i, 0)),
        ],
        out_specs=[pl.BlockSpec((MB, dh), lambda i: (i, 0))] * 2,
        out_shape=[jax.ShapeDtypeStruct((N, dh), jnp.float32)] * 2,
    )(x, W1, degt)


def _tc_mid(agg_a, agg_b, xs_a, xs_b, degt, b, W, din, dout):
    dhi, dho = din // 2, dout // 2

    def body(aa, ab, xa, xb, deg_ref, b_ref, w_ref, oa_ref, ob_ref):
        dis = _dis_of(deg_ref)
        aggf = jnp.concatenate([aa[...], ab[...]], axis=1)
        xsf = jnp.concatenate([xa[...], xb[...]], axis=1)
        h = jax.nn.relu(dis * (aggf + xsf) + b_ref[...])
        xw = jnp.dot(h, w_ref[...], preferred_element_type=jnp.float32)
        xs2 = xw * dis
        oa_ref[...] = xs2[:, :dho]
        ob_ref[...] = xs2[:, dho:]

    return pl.pallas_call(
        body,
        grid=(GRID,),
        in_specs=[
            pl.BlockSpec((MB, dhi), lambda i: (i, 0)),
            pl.BlockSpec((MB, dhi), lambda i: (i, 0)),
            pl.BlockSpec((MB, dhi), lambda i: (i, 0)),
            pl.BlockSpec((MB, dhi), lambda i: (i, 0)),
            pl.BlockSpec((MB, 2), lambda i: (i, 0)),
            pl.BlockSpec((1, din), lambda i: (0, 0)),
            pl.BlockSpec((din, dout), lambda i: (0, 0)),
        ],
        out_specs=[pl.BlockSpec((MB, dho), lambda i: (i, 0))] * 2,
        out_shape=[jax.ShapeDtypeStruct((N, dho), jnp.float32)] * 2,
    )(agg_a, agg_b, xs_a, xs_b, degt, b, W)


def _tc_final(agg_a, agg_b, xs_a, xs_b, degt, b3, batch2, Wl, bl):
    dhi = H3 // 2

    def body(aa, ab, xa, xb, deg_ref, b_ref, bt_ref, wl_ref, bl_ref,
             out_ref, sums_ref, cnts_ref):
        i = pl.program_id(0)

        @pl.when(i == 0)
        def _():
            sums_ref[...] = jnp.zeros_like(sums_ref)
            cnts_ref[...] = jnp.zeros_like(cnts_ref)

        dis = _dis_of(deg_ref)
        aggf = jnp.concatenate([aa[...], ab[...]], axis=1)
        xsf = jnp.concatenate([xa[...], xb[...]], axis=1)
        h = dis * (aggf + xsf) + b_ref[...]
        y = jnp.dot(h, wl_ref[...], preferred_element_type=jnp.float32)

        validr = (lax.broadcasted_iota(jnp.int32, (MB, 1), 0) + i * MB) < N
        validc = (lax.broadcasted_iota(jnp.int32, (1, MB), 1) + i * MB) < N
        ym = jnp.where(validr, y, 0.0)
        oh = (lax.broadcasted_iota(jnp.int32, (G, MB), 0) == bt_ref[...]).astype(jnp.float32)
        ohm = jnp.where(validc, oh, 0.0)
        sums_ref[...] += jnp.dot(ohm, ym, preferred_element_type=jnp.float32)
        cnts_ref[...] += jnp.sum(ohm, axis=1, keepdims=True)

        @pl.when(i == GRID - 1)
        def _():
            out_ref[...] = (sums_ref[...] / jnp.maximum(cnts_ref[...], 1.0)
                            + bl_ref[...])

    out, _, _ = pl.pallas_call(
        body,
        grid=(GRID,),
        in_specs=[
            pl.BlockSpec((MB, dhi), lambda i: (i, 0)),
            pl.BlockSpec((MB, dhi), lambda i: (i, 0)),
            pl.BlockSpec((MB, dhi), lambda i: (i, 0)),
            pl.BlockSpec((MB, dhi), lambda i: (i, 0)),
            pl.BlockSpec((MB, 2), lambda i: (i, 0)),
            pl.BlockSpec((1, H3), lambda i: (0, 0)),
            pl.BlockSpec((1, MB), lambda i: (0, i)),
            pl.BlockSpec((H3, D_OUT), lambda i: (0, 0)),
            pl.BlockSpec((1, D_OUT), lambda i: (0, 0)),
        ],
        out_specs=[
            pl.BlockSpec((G, D_OUT), lambda i: (0, 0)),
            pl.BlockSpec((G, D_OUT), lambda i: (0, 0)),
            pl.BlockSpec((G, 1), lambda i: (0, 0)),
        ],
        out_shape=[
            jax.ShapeDtypeStruct((G, D_OUT), jnp.float32),
            jax.ShapeDtypeStruct((G, D_OUT), jnp.float32),
            jax.ShapeDtypeStruct((G, 1), jnp.float32),
        ],
    )(agg_a, agg_b, xs_a, xs_b, degt, b3, batch2, Wl, bl)
    return out


def kernel(x, edge_index, edge_weight, batch, W1, b1, W2, b2, W3, b3, Wl, bl):
    r = edge_index[0]
    c = edge_index[1]
    r_agg = r.reshape(NS, NCH_AGG, CH)
    c_agg = c.reshape(NS, NCH_AGG, CH)
    w_agg = edge_weight.reshape(NS, EC_AGG)
    c_deg = c.reshape(NS * NC, NCH_DEG, CH)
    w_deg = edge_weight.reshape(NS * NC, NCH_DEG, CH)

    degp = _deg_call(c_deg, w_deg)
    degt = degp.reshape(NC, N).T

    xs1a, xs1b = _tc1(x, W1, degt)
    agg1a, agg1b = _agg[H1 // 2](xs1a, xs1b, r_agg, c_agg, w_agg)
    xs2a, xs2b = _tc_mid(agg1a, agg1b, xs1a, xs1b, degt,
                         b1.reshape(1, H1), W2, H1, H2)
    agg2a, agg2b = _agg[H2 // 2](xs2a, xs2b, r_agg, c_agg, w_agg)
    xs3a, xs3b = _tc_mid(agg2a, agg2b, xs2a, xs2b, degt,
                         b2.reshape(1, H2), W3, H2, H3)
    agg3a, agg3b = _agg[H3 // 2](xs3a, xs3b, r_agg, c_agg, w_agg)
    out = _tc_final(agg3a, agg3b, xs3a, xs3b, degt,
                    b3.reshape(1, H3), batch.reshape(1, N), Wl,
                    bl.reshape(1, D_OUT))
    return out


# speedup vs baseline: 9.7315x; 1.2097x over previous
import functools

import jax
import jax.numpy as jnp
from jax import lax
from jax.experimental import pallas as pl
from jax.experimental.pallas import tpu as pltpu
from jax.experimental.pallas import tpu_sc as plsc

N = 10000
E = 160000
G = 64
D_IN = 1056
H1, H2, H3, D_OUT = 256, 128, 64, 3

NC, NS = 2, 16
CH = 125
EC_AGG = E // NS
NCH_AGG = EC_AGG // CH
EC_DEG = E // (NS * NC)
NCH_DEG = EC_DEG // CH
ROWS_FULL = 640
ROWS_LAST = N - (NS - 1) * ROWS_FULL
ZROWS = 80

MB = 512
GRID = (N + MB - 1) // MB

_mesh = plsc.VectorSubcoreMesh(
    core_axis_name="c", subcore_axis_name="s", num_cores=NC, num_subcores=NS)
_sc_params = pltpu.CompilerParams(
    needs_layout_passes=False, use_tc_tiling_on_sc=False)


def _deg_body(c_hbm, w_hbm, out_hbm, c_v, w_v, zb, deg_sh):
    cid = lax.axis_index("c")
    sid = lax.axis_index("s")
    wid = cid * NS + sid
    pltpu.sync_copy(c_hbm.at[wid], c_v)
    pltpu.sync_copy(w_hbm.at[wid], w_v)
    for i in range(ROWS_FULL // 16):
        zb[pl.ds(i * 16, 16)] = jnp.zeros((16,), jnp.float32)
    row0 = sid * ROWS_FULL

    @pl.when(sid < NS - 1)
    def _():
        pltpu.sync_copy(zb, deg_sh.at[pl.ds(row0, ROWS_FULL)])

    @pl.when(sid == NS - 1)
    def _():
        pltpu.sync_copy(zb.at[pl.ds(0, ROWS_LAST)], deg_sh.at[pl.ds(row0, ROWS_LAST)])

    plsc.subcore_barrier()

    @pl.loop(0, NCH_DEG)
    def _scatter(j):
        pltpu.sync_copy(w_v.at[j], deg_sh.at[c_v.at[j]], add=True)

    plsc.subcore_barrier()
    base = cid * N + row0

    @pl.when(sid < NS - 1)
    def _():
        pltpu.sync_copy(deg_sh.at[pl.ds(row0, ROWS_FULL)], zb)
        pltpu.sync_copy(zb, out_hbm.at[pl.ds(base, ROWS_FULL)])

    @pl.when(sid == NS - 1)
    def _():
        pltpu.sync_copy(deg_sh.at[pl.ds(row0, ROWS_LAST)], zb.at[pl.ds(0, ROWS_LAST)])
        pltpu.sync_copy(zb.at[pl.ds(0, ROWS_LAST)], out_hbm.at[pl.ds(base, ROWS_LAST)])


_deg_call = pl.kernel(
    _deg_body,
    out_type=jax.ShapeDtypeStruct((NC * N,), jnp.float32),
    mesh=_mesh,
    compiler_params=_sc_params,
    scratch_types=[
        pltpu.VMEM((NCH_DEG, CH), jnp.int32),
        pltpu.VMEM((NCH_DEG, CH), jnp.float32),
        pltpu.VMEM((ROWS_FULL,), jnp.float32),
        pltpu.MemorySpace.VMEM_SHARED((N,), jnp.float32),
    ],
)


def _make_agg(dh):
    def scale_rows(buf, w_v, j):
        @pl.loop(0, CH, unroll=5)
        def _row(e):
            fv = jnp.full((16,), j * CH + e, jnp.int32)
            wv = plsc.load_gather(w_v, [fv])
            for k in range(dh // 16):
                sl = pl.ds(k * 16, 16)
                buf[e, sl] = buf[e, sl] * wv

    def body(xs_a, xs_b, rc_hbm, w_hbm, out_a, out_b,
             rc_v, w_v, buf0, buf1, agg_sh,
             isem0, isem1, gsem0, gsem1):
        cid = lax.axis_index("c")
        sid = lax.axis_index("s")
        pltpu.sync_copy(w_hbm.at[sid], w_v)

        @pl.loop(0, CH)
        def _z(i):
            for k in range(dh // 16):
                buf0[i, pl.ds(k * 16, 16)] = jnp.zeros((16,), jnp.float32)

        zblk = buf0.at[pl.ds(0, ZROWS)]
        row0 = sid * ROWS_FULL

        @pl.when(sid < NS - 1)
        def _():
            @pl.loop(0, ROWS_FULL // ZROWS)
            def _c(i):
                pltpu.sync_copy(zblk, agg_sh.at[pl.ds(row0 + i * ZROWS, ZROWS)])

        @pl.when(sid == NS - 1)
        def _():
            @pl.loop(0, ROWS_LAST // ZROWS)
            def _c(i):
                pltpu.sync_copy(zblk, agg_sh.at[pl.ds(row0 + i * ZROWS, ZROWS)])

        plsc.subcore_barrier()

        def run(xs_hbm):
            pltpu.sync_copy(rc_hbm.at[sid, 0], rc_v.at[0])
            g0 = pltpu.make_async_copy(xs_hbm.at[rc_v.at[0, 0]], buf0, gsem0)
            g0.start()
            pltpu.make_async_copy(rc_hbm.at[sid, 1], rc_v.at[1], isem1).start()

            @pl.loop(0, NCH_AGG // 2)
            def _outer(t):
                j0 = 2 * t
                j1 = j0 + 1
                pltpu.make_async_copy(xs_hbm.at[rc_v.at[0, 0]], buf0,
                                      gsem0).wait()
                pltpu.make_async_copy(rc_hbm.at[sid, 0], rc_v.at[1],
                                      isem1).wait()
                pltpu.make_async_copy(xs_hbm.at[rc_v.at[1, 0]], buf1,
                                      gsem1).start()
                scale_rows(buf0, w_v, j0)
                pltpu.sync_copy(buf0, agg_sh.at[rc_v.at[0, 1]], add=True)

                @pl.when(t < NCH_AGG // 2 - 1)
                def _():
                    pltpu.make_async_copy(rc_hbm.at[sid, j0 + 2], rc_v.at[0],
                                          isem0).start()

                pltpu.make_async_copy(xs_hbm.at[rc_v.at[1, 0]], buf1,
                                      gsem1).wait()

                @pl.when(t < NCH_AGG // 2 - 1)
                def _():
                    pltpu.make_async_copy(rc_hbm.at[sid, 0], rc_v.at[0],
                                          isem0).wait()
                    pltpu.make_async_copy(xs_hbm.at[rc_v.at[0, 0]], buf0,
                                          gsem0).start()

                scale_rows(buf1, w_v, j1)
                pltpu.sync_copy(buf1, agg_sh.at[rc_v.at[1, 1]], add=True)

                @pl.when(t < NCH_AGG // 2 - 1)
                def _():
                    pltpu.make_async_copy(rc_hbm.at[sid, j1 + 2], rc_v.at[1],
                                          isem1).start()

        @pl.when(cid == 0)
        def _():
            run(xs_a)

        @pl.when(cid == 1)
        def _():
            run(xs_b)

        plsc.subcore_barrier()
        zblk2 = buf0.at[pl.ds(0, ZROWS)]

        def drain(out_hbm):
            @pl.when(sid < NS - 1)
            def _():
                @pl.loop(0, ROWS_FULL // ZROWS)
                def _d(i):
                    pltpu.sync_copy(agg_sh.at[pl.ds(row0 + i * ZROWS, ZROWS)], zblk2)
                    pltpu.sync_copy(zblk2, out_hbm.at[pl.ds(row0 + i * ZROWS, ZROWS)])

            @pl.when(sid == NS - 1)
            def _():
                @pl.loop(0, ROWS_LAST // ZROWS)
                def _d(i):
                    pltpu.sync_copy(agg_sh.at[pl.ds(row0 + i * ZROWS, ZROWS)], zblk2)
                    pltpu.sync_copy(zblk2, out_hbm.at[pl.ds(row0 + i * ZROWS, ZROWS)])

        @pl.when(cid == 0)
        def _():
            drain(out_a)

        @pl.when(cid == 1)
        def _():
            drain(out_b)

    return pl.kernel(
        body,
        out_type=[jax.ShapeDtypeStruct((N, dh), jnp.float32)] * 2,
        mesh=_mesh,
        compiler_params=_sc_params,
        scratch_types=[
            pltpu.VMEM((2, 2, CH), jnp.int32),
            pltpu.VMEM((EC_AGG,), jnp.float32),
            pltpu.VMEM((CH, dh), jnp.float32),
            pltpu.VMEM((CH, dh), jnp.float32),
            pltpu.MemorySpace.VMEM_SHARED((N, dh), jnp.float32),
            pltpu.SemaphoreType.DMA,
            pltpu.SemaphoreType.DMA,
            pltpu.SemaphoreType.DMA,
            pltpu.SemaphoreType.DMA,
        ],
    )


_agg = {dh: _make_agg(dh) for dh in (H1 // 2, H2 // 2, H3 // 2)}


def _dis_of(deg_ref):
    d = deg_ref[...]
    s = d[:, 0:1] + d[:, 1:2] + 1.0
    return jnp.where(s > 0, lax.rsqrt(s), 0.0)


def _tc1(x, W1, degt):
    dh = H1 // 2

    def body(x_ref, w_ref, deg_ref, oa_ref, ob_ref):
        dis = _dis_of(deg_ref)
        xw = jnp.dot(x_ref[...], w_ref[...], preferred_element_type=jnp.float32)
        xs = xw * dis
        oa_ref[...] = xs[:, :dh]
        ob_ref[...] = xs[:, dh:]

    return pl.pallas_call(
        body,
        grid=(GRID,),
        in_specs=[
            pl.BlockSpec((MB, D_IN), lambda i: (i, 0)),
            pl.BlockSpec((D_IN, H1), lambda i: (0, 0)),
            pl.BlockSpec((MB, 2), lambda i: (i, 0)),
        ],
        out_specs=[pl.BlockSpec((MB, dh), lambda i: (i, 0))] * 2,
        out_shape=[jax.ShapeDtypeStruct((N, dh), jnp.float32)] * 2,
    )(x, W1, degt)


def _tc_mid(agg_a, agg_b, xs_a, xs_b, degt, b, W, din, dout):
    dhi, dho = din // 2, dout // 2

    def body(aa, ab, xa, xb, deg_ref, b_ref, w_ref, oa_ref, ob_ref):
        dis = _dis_of(deg_ref)
        aggf = jnp.concatenate([aa[...], ab[...]], axis=1)
        xsf = jnp.concatenate([xa[...], xb[...]], axis=1)
        h = jax.nn.relu(dis * (aggf + xsf) + b_ref[...])
        xw = jnp.dot(h, w_ref[...], preferred_element_type=jnp.float32)
        xs2 = xw * dis
        oa_ref[...] = xs2[:, :dho]
        ob_ref[...] = xs2[:, dho:]

    return pl.pallas_call(
        body,
        grid=(GRID,),
        in_specs=[
            pl.BlockSpec((MB, dhi), lambda i: (i, 0)),
            pl.BlockSpec((MB, dhi), lambda i: (i, 0)),
            pl.BlockSpec((MB, dhi), lambda i: (i, 0)),
            pl.BlockSpec((MB, dhi), lambda i: (i, 0)),
            pl.BlockSpec((MB, 2), lambda i: (i, 0)),
            pl.BlockSpec((1, din), lambda i: (0, 0)),
            pl.BlockSpec((din, dout), lambda i: (0, 0)),
        ],
        out_specs=[pl.BlockSpec((MB, dho), lambda i: (i, 0))] * 2,
        out_shape=[jax.ShapeDtypeStruct((N, dho), jnp.float32)] * 2,
    )(agg_a, agg_b, xs_a, xs_b, degt, b, W)


def _tc_final(agg_a, agg_b, xs_a, xs_b, degt, b3, batch2, Wl, bl):
    dhi = H3 // 2

    def body(aa, ab, xa, xb, deg_ref, b_ref, bt_ref, wl_ref, bl_ref,
             out_ref, sums_ref, cnts_ref):
        i = pl.program_id(0)

        @pl.when(i == 0)
        def _():
            sums_ref[...] = jnp.zeros_like(sums_ref)
            cnts_ref[...] = jnp.zeros_like(cnts_ref)

        dis = _dis_of(deg_ref)
        aggf = jnp.concatenate([aa[...], ab[...]], axis=1)
        xsf = jnp.concatenate([xa[...], xb[...]], axis=1)
        h = dis * (aggf + xsf) + b_ref[...]
        y = jnp.dot(h, wl_ref[...], preferred_element_type=jnp.float32)

        validr = (lax.broadcasted_iota(jnp.int32, (MB, 1), 0) + i * MB) < N
        validc = (lax.broadcasted_iota(jnp.int32, (1, MB), 1) + i * MB) < N
        ym = jnp.where(validr, y, 0.0)
        oh = (lax.broadcasted_iota(jnp.int32, (G, MB), 0) == bt_ref[...]).astype(jnp.float32)
        ohm = jnp.where(validc, oh, 0.0)
        sums_ref[...] += jnp.dot(ohm, ym, preferred_element_type=jnp.float32)
        cnts_ref[...] += jnp.sum(ohm, axis=1, keepdims=True)

        @pl.when(i == GRID - 1)
        def _():
            out_ref[...] = (sums_ref[...] / jnp.maximum(cnts_ref[...], 1.0)
                            + bl_ref[...])

    out, _, _ = pl.pallas_call(
        body,
        grid=(GRID,),
        in_specs=[
            pl.BlockSpec((MB, dhi), lambda i: (i, 0)),
            pl.BlockSpec((MB, dhi), lambda i: (i, 0)),
            pl.BlockSpec((MB, dhi), lambda i: (i, 0)),
            pl.BlockSpec((MB, dhi), lambda i: (i, 0)),
            pl.BlockSpec((MB, 2), lambda i: (i, 0)),
            pl.BlockSpec((1, H3), lambda i: (0, 0)),
            pl.BlockSpec((1, MB), lambda i: (0, i)),
            pl.BlockSpec((H3, D_OUT), lambda i: (0, 0)),
            pl.BlockSpec((1, D_OUT), lambda i: (0, 0)),
        ],
        out_specs=[
            pl.BlockSpec((G, D_OUT), lambda i: (0, 0)),
            pl.BlockSpec((G, D_OUT), lambda i: (0, 0)),
            pl.BlockSpec((G, 1), lambda i: (0, 0)),
        ],
        out_shape=[
            jax.ShapeDtypeStruct((G, D_OUT), jnp.float32),
            jax.ShapeDtypeStruct((G, D_OUT), jnp.float32),
            jax.ShapeDtypeStruct((G, 1), jnp.float32),
        ],
    )(agg_a, agg_b, xs_a, xs_b, degt, b3, batch2, Wl, bl)
    return out


def kernel(x, edge_index, edge_weight, batch, W1, b1, W2, b2, W3, b3, Wl, bl):
    r = edge_index[0]
    c = edge_index[1]
    rc_agg = jnp.stack(
        [r.reshape(NS, NCH_AGG, CH), c.reshape(NS, NCH_AGG, CH)], axis=2)
    w_agg = edge_weight.reshape(NS, EC_AGG)
    c_deg = c.reshape(NS * NC, NCH_DEG, CH)
    w_deg = edge_weight.reshape(NS * NC, NCH_DEG, CH)

    degp = _deg_call(c_deg, w_deg)
    degt = degp.reshape(NC, N).T

    xs1a, xs1b = _tc1(x, W1, degt)
    agg1a, agg1b = _agg[H1 // 2](xs1a, xs1b, rc_agg, w_agg)
    xs2a, xs2b = _tc_mid(agg1a, agg1b, xs1a, xs1b, degt,
                         b1.reshape(1, H1), W2, H1, H2)
    agg2a, agg2b = _agg[H2 // 2](xs2a, xs2b, rc_agg, w_agg)
    xs3a, xs3b = _tc_mid(agg2a, agg2b, xs2a, xs2b, degt,
                         b2.reshape(1, H2), W3, H2, H3)
    agg3a, agg3b = _agg[H3 // 2](xs3a, xs3b, rc_agg, w_agg)
    out = _tc_final(agg3a, agg3b, xs3a, xs3b, degt,
                    b3.reshape(1, H3), batch.reshape(1, N), Wl,
                    bl.reshape(1, D_OUT))
    return out
